# Initial kernel scaffold; baseline (speedup 1.0000x reference)
import jax
import jax.numpy as jnp
from jax.experimental import pallas as pl


def kernel(xyz, params):
    raise NotImplementedError("write your pallas kernel here")


# speedup vs baseline: 5.0230x; 5.0230x over previous
import functools

import jax
import jax.numpy as jnp
import numpy as np
from jax.experimental import pallas as pl

BN_SCALE = float(1.0 / np.sqrt(1.0 + 1e-5))
BIG = 1e9


def _dot(a, b):
    return jnp.dot(a, b, precision=jax.lax.Precision.HIGHEST,
                   preferred_element_type=jnp.float32)


def _dotbf(a, b):
    return jnp.dot(a.astype(jnp.bfloat16), b.astype(jnp.bfloat16),
                   preferred_element_type=jnp.float32)


def _iota_f32(shape, dim):
    return jax.lax.broadcasted_iota(jnp.int32, shape, dim).astype(jnp.float32)


def _fps_body(npoint, x_ref, o_ref):
    x = x_ref[...]
    B, _, N = x.shape
    iota_n = _iota_f32((B, 1, N), 2)
    iota_s = _iota_f32((B, 1, npoint), 2)

    def body(i, state):
        dist, far, acc = state
        oh = jnp.where(iota_n == far, 1.0, 0.0)
        centroid = jnp.sum(x * oh, axis=2, keepdims=True)
        acc = jnp.where(iota_s == i.astype(jnp.float32), centroid, acc)
        d = jnp.sum((x - centroid) ** 2, axis=1, keepdims=True)
        dist = jnp.minimum(dist, d)
        m = jnp.max(dist, axis=2, keepdims=True)
        far = jnp.min(jnp.where(dist == m, iota_n, BIG), axis=2, keepdims=True)
        return dist, far, acc

    init = (jnp.full((B, 1, N), 1e10, jnp.float32),
            jnp.zeros((B, 1, 1), jnp.float32),
            jnp.zeros((B, 3, npoint), jnp.float32))
    _, _, acc = jax.lax.fori_loop(0, npoint, body, init)
    o_ref[...] = acc


def _fps(x_t, npoint):
    B, _, N = x_t.shape
    return pl.pallas_call(
        functools.partial(_fps_body, npoint),
        out_shape=jax.ShapeDtypeStruct((B, 3, npoint), jnp.float32),
    )(x_t)


def _sa_body(radius, K, Cf, xyz_first, refs):
    if Cf > 0:
        (xt_ref, xg_ref, c_ref, f_ref,
         wx_ref, wf_ref, b1_ref, w2_ref, b2_ref, w3_ref, b3_ref, o_ref) = refs
    else:
        (xt_ref, xg_ref, c_ref,
         wx_ref, b1_ref, w2_ref, b2_ref, w3_ref, b3_ref, o_ref) = refs
        f_ref = wf_ref = None

    xt = xt_ref[0]
    xg = xg_ref[0]
    c = c_ref[0]
    Ts, N = c.shape[0], xt.shape[1]

    cx = _dotbf(c, xt)
    sqc = jnp.sum(c * c, axis=1, keepdims=True)
    sqx = jnp.sum(xt * xt, axis=0, keepdims=True)
    d = (sqc + sqx) - 2.0 * cx

    iota = _iota_f32((Ts, N), 1)
    key = jnp.where(d > radius * radius, BIG, iota)

    idxs = []
    for _ in range(K):
        cur = jnp.min(key, axis=1, keepdims=True)
        idxs.append(cur)
        key = jnp.where(key == cur, BIG, key)
    idx0 = jnp.where(idxs[0] < BIG, idxs[0], jnp.float32(N - 1))
    idxs = [idx0] + [jnp.where(cur < BIG, cur, idx0) for cur in idxs[1:]]

    acc = None
    for idx_k in idxs:
        oh = jnp.where(iota == idx_k, 1.0, 0.0)
        gx = _dot(oh, xg) - c
        h = _dotbf(gx, wx_ref[...])
        if f_ref is not None:
            gf = _dot(oh, f_ref[0])
            h = h + _dotbf(gf, wf_ref[...])
        h = jax.nn.relu((h + b1_ref[...]) * BN_SCALE)
        h = jax.nn.relu((_dotbf(h, w2_ref[...]) + b2_ref[...]) * BN_SCALE)
        h = jax.nn.relu((_dotbf(h, w3_ref[...]) + b3_ref[...]) * BN_SCALE)
        acc = h if acc is None else jnp.maximum(acc, h)
    o_ref[0] = acc


def _sa_branch(x_t, x_g, c_g, feat, layers, radius, K, Ts, xyz_first):
    B, _, N = x_t.shape
    S = c_g.shape[1]
    (W1, b1), (W2, b2), (W3, b3) = layers
    Cf = feat.shape[2] if feat is not None else 0
    W1t = W1.T
    if Cf > 0:
        if xyz_first:
            wx, wf = W1t[:3], W1t[3:]
        else:
            wf, wx = W1t[:Cf], W1t[Cf:]
    else:
        wx, wf = W1t, None
    C1, C2, C3 = W1.shape[0], W2.shape[0], W3.shape[0]

    grid = (B, S // Ts)
    full = lambda shape: pl.BlockSpec(shape, lambda b, t: (0,) * len(shape))
    in_specs = [
        pl.BlockSpec((1, 3, N), lambda b, t: (b, 0, 0)),
        pl.BlockSpec((1, N, 3), lambda b, t: (b, 0, 0)),
        pl.BlockSpec((1, Ts, 3), lambda b, t: (b, t, 0)),
    ]
    args = [x_t, x_g, c_g]
    if Cf > 0:
        in_specs.append(pl.BlockSpec((1, N, Cf), lambda b, t: (b, 0, 0)))
        args.append(feat)
        in_specs += [full((3, C1)), full((Cf, C1)), full((1, C1)),
                     full((C1, C2)), full((1, C2)), full((C2, C3)), full((1, C3))]
        args += [wx, wf, b1.reshape(1, C1), W2.T, b2.reshape(1, C2),
                 W3.T, b3.reshape(1, C3)]
    else:
        in_specs += [full((3, C1)), full((1, C1)),
                     full((C1, C2)), full((1, C2)), full((C2, C3)), full((1, C3))]
        args += [wx, b1.reshape(1, C1), W2.T, b2.reshape(1, C2),
                 W3.T, b3.reshape(1, C3)]

    body = lambda *refs: _sa_body(radius, K, Cf, xyz_first, refs)
    return pl.pallas_call(
        body,
        grid=grid,
        in_specs=in_specs,
        out_specs=pl.BlockSpec((1, Ts, C3), lambda b, t: (b, t, 0)),
        out_shape=jax.ShapeDtypeStruct((B, S, C3), jnp.float32),
    )(*args)


def _fp_body(C1, head, refs, *, nlayers):
    i = 0
    x1t_ref = refs[i]; i += 1
    x1g_ref = refs[i]; i += 1
    x2t_ref = refs[i]; i += 1
    p1_ref = None
    if C1 > 0:
        p1_ref = refs[i]; i += 1
    p2_ref = refs[i]; i += 1
    wlist = []
    for _ in range(nlayers):
        wlist.append((refs[i], refs[i + 1])); i += 2
    if head:
        wc1, bc1, wc2, bc2 = refs[i:i + 4]; i += 4
        o_ref, l_ref = refs[i], refs[i + 1]
    else:
        o_ref = refs[i]

    x1 = x1g_ref[0]
    x2t = x2t_ref[0]
    Ts, S2 = x1.shape[0], x2t.shape[1]

    cx = _dotbf(x1, x2t)
    sq1 = jnp.sum(x1 * x1, axis=1, keepdims=True)
    sq2 = jnp.sum(x2t * x2t, axis=0, keepdims=True)
    d = (sq1 + sq2) - 2.0 * cx

    iota = _iota_f32((Ts, S2), 1)
    w_sp = jnp.zeros((Ts, S2), jnp.float32)
    rsum = jnp.zeros((Ts, 1), jnp.float32)
    dd = d
    for _ in range(3):
        cur = jnp.min(dd, axis=1, keepdims=True)
        slv = dd == cur
        pidx = jnp.min(jnp.where(slv, iota, BIG), axis=1, keepdims=True)
        pos = jnp.logical_and(slv, iota == pidx)
        r = 1.0 / (cur + 1e-8)
        w_sp = w_sp + jnp.where(pos, r, 0.0)
        rsum = rsum + r
        dd = jnp.where(pos, BIG, dd)
    w_sp = w_sp / rsum

    h = _dot(w_sp, p2_ref[0])
    for li, (w_ref, b_ref) in enumerate(wlist):
        w = w_ref[...]
        if li == 0 and C1 > 0:
            pre = (_dotbf(p1_ref[0], w[:C1])
                   + _dotbf(h, w[C1:]))
        else:
            pre = _dotbf(h, w)
        h = jax.nn.relu((pre + b_ref[...]) * BN_SCALE)
    o_ref[0] = h
    if head:
        ft = jax.nn.relu((_dotbf(h, wc1[...])
                          + bc1[...]) * BN_SCALE)
        l_ref[0] = _dotbf(ft, wc2[...]) + bc2[...]


def _fp(x1_t, x1_g, x2_t, points1, points2, layers, Ts, head=None):
    B, _, N1 = x1_t.shape
    S2 = x2_t.shape[2]
    C1 = points1.shape[2] if points1 is not None else 0
    C2 = points2.shape[2]

    grid = (B, N1 // Ts)
    full = lambda shape: pl.BlockSpec(shape, lambda b, t: (0,) * len(shape))
    in_specs = [
        pl.BlockSpec((1, 3, N1), lambda b, t: (b, 0, 0)),
        pl.BlockSpec((1, Ts, 3), lambda b, t: (b, t, 0)),
        pl.BlockSpec((1, 3, S2), lambda b, t: (b, 0, 0)),
    ]
    args = [x1_t, x1_g, x2_t]
    if C1 > 0:
        in_specs.append(pl.BlockSpec((1, Ts, C1), lambda b, t: (b, t, 0)))
        args.append(points1)
    in_specs.append(pl.BlockSpec((1, S2, C2), lambda b, t: (b, 0, 0)))
    args.append(points2)
    cin = C1 + C2
    for (W, b) in layers:
        cout = W.shape[0]
        in_specs += [full((cin, cout)), full((1, cout))]
        args += [W.T, b.reshape(1, cout)]
        cin = cout
    out_shapes = [jax.ShapeDtypeStruct((B, N1, cin), jnp.float32)]
    out_specs = [pl.BlockSpec((1, Ts, cin), lambda b, t: (b, t, 0))]
    if head is not None:
        (Wc1, bc1), (Wc2, bc2) = head
        in_specs += [full((cin, Wc1.shape[0])), full((1, Wc1.shape[0])),
                     full((Wc1.shape[0], 1)), full((1, 1))]
        args += [Wc1.T, bc1.reshape(1, -1), Wc2.T, bc2.reshape(1, 1)]
        out_shapes.append(jax.ShapeDtypeStruct((B, N1, 1), jnp.float32))
        out_specs.append(pl.BlockSpec((1, Ts, 1), lambda b, t: (b, t, 0)))

    body = functools.partial(_fp_body, C1, head is not None, nlayers=len(layers))
    outs = pl.pallas_call(
        lambda *refs: body(refs),
        grid=grid,
        in_specs=in_specs,
        out_specs=out_specs,
        out_shape=out_shapes,
    )(*args)
    return outs if head is not None else outs[0]


def _sa_msg(x_t, x_g, feat, npoint, radii, Ks, branches, Ts):
    c_t = _fps(x_t, npoint)
    c_g = jnp.transpose(c_t, (0, 2, 1))
    outs = [
        _sa_branch(x_t, x_g, c_g, feat, layers, r, K, Ts, xyz_first=False)
        for r, K, layers in zip(radii, Ks, branches)
    ]
    return c_t, c_g, jnp.concatenate(outs, axis=-1)


def kernel(xyz, params):
    x0_g = xyz[:, :, 0:3]
    x0_t = jnp.transpose(x0_g, (0, 2, 1))

    l1_t, l1_g, l1_p = _sa_msg(x0_t, x0_g, None, 640, [0.05, 0.1], [16, 32],
                               params["sa1"], Ts=128)
    l2_t, l2_g, l2_p = _sa_msg(l1_t, l1_g, l1_p, 320, [0.1, 0.2], [16, 32],
                               params["sa2"], Ts=320)
    l3_t, l3_g, l3_p = _sa_msg(l2_t, l2_g, l2_p, 160, [0.2, 0.4], [16, 32],
                               params["sa3"], Ts=160)

    l4_t = _fps(l3_t, 40)
    l4_g = jnp.transpose(l4_t, (0, 2, 1))
    l4_p = _sa_branch(l3_t, l3_g, l4_g, l3_p, params["sa4"], 0.4, 32, 40,
                      xyz_first=True)

    l3_p = _fp(l3_t, l3_g, l4_t, l3_p, l4_p, params["fp4"], Ts=160)
    l2_p = _fp(l2_t, l2_g, l3_t, l2_p, l3_p, params["fp3"], Ts=320)
    l1_p = _fp(l1_t, l1_g, l2_t, l1_p, l2_p, params["fp2"], Ts=640)
    l0_feat, logits = _fp(x0_t, x0_g, l1_t, None, l1_p, params["fp1"], Ts=512,
                          head=(params["conv1"], params["conv2"]))

    return logits[..., 0], jnp.transpose(l0_feat, (0, 2, 1))
